# Initial kernel scaffold; baseline (speedup 1.0000x reference)
import jax
import jax.numpy as jnp
from jax.experimental import pallas as pl


def kernel(x, edge_index, W, b):
    raise NotImplementedError("write your pallas kernel here")


# speedup vs baseline: 217.1969x; 217.1969x over previous
import functools

import jax
import jax.numpy as jnp
from jax import lax
from jax.experimental import pallas as pl
from jax.experimental.pallas import tpu as pltpu
from jax.experimental.pallas import tpu_sc as plsc

NC = 2
NS = 16
NW = NC * NS
L = 16

N_NODES = 100000
NPAD = 102400
SLICE = NPAD // NS
ROWS_PT = 1568
ROWS = ROWS_PT * NW
EPAD = ROWS * 128
R = 16
NMACRO = ROWS_PT // R
PAD_NODE = NPAD - 1

_mesh = plsc.VectorSubcoreMesh(core_axis_name="c", subcore_axis_name="s")
_sc_params = pltpu.CompilerParams(needs_layout_passes=False)


def _zero_vmem(buf, nwords):
    z = jnp.zeros((L,), jnp.float32)

    def body(i, _):
        buf[pl.ds(i * L, L)] = z
        return 0

    lax.fori_loop(0, nwords // L, body, 0)


@functools.partial(
    pl.kernel,
    out_type=jax.ShapeDtypeStruct((NW, SLICE), jnp.float32),
    mesh=_mesh,
    scratch_types=[
        pltpu.VMEM((R, 128), jnp.int32),
        pltpu.VMEM((128,), jnp.float32),
        pltpu.VMEM((SLICE,), jnp.float32),
        pltpu.VMEM_SHARED((NPAD,), jnp.float32),
    ],
    compiler_params=_sc_params,
)
def _degree_kernel(dst_hbm, out_hbm, idx_v, ones_v, bounce_v, acc_sh):
    c = lax.axis_index("c")
    s = lax.axis_index("s")
    wid = c * NS + s

    one = jnp.ones((L,), jnp.float32)
    for k in range(128 // L):
        ones_v[pl.ds(k * L, L)] = one

    _zero_vmem(bounce_v, SLICE)
    pltpu.sync_copy(bounce_v, acc_sh.at[pl.ds(s * SLICE, SLICE)])
    plsc.subcore_barrier()

    row0 = wid * ROWS_PT

    def macro(m, _):
        pltpu.sync_copy(dst_hbm.at[pl.ds(row0 + m * R, R)], idx_v)
        for j in range(R):
            pltpu.sync_copy(ones_v, acc_sh.at[idx_v.at[j]], add=True)
        return 0

    lax.fori_loop(0, NMACRO, macro, 0)
    plsc.subcore_barrier()

    pltpu.sync_copy(acc_sh.at[pl.ds(s * SLICE, SLICE)], bounce_v)
    pltpu.sync_copy(bounce_v, out_hbm.at[wid])


@functools.partial(
    pl.kernel,
    out_type=jax.ShapeDtypeStruct((NW, SLICE), jnp.float32),
    mesh=_mesh,
    scratch_types=[
        pltpu.VMEM((R * 128,), jnp.int32),
        pltpu.VMEM((R, 128), jnp.int32),
        pltpu.VMEM((128,), jnp.float32),
        pltpu.VMEM((SLICE,), jnp.float32),
        pltpu.VMEM((NPAD,), jnp.float32),
        pltpu.VMEM_SHARED((NPAD,), jnp.float32),
    ],
    compiler_params=_sc_params,
)
def _scatter_kernel(src_hbm, dst_hbm, y_hbm, out_hbm,
                    src_v, dstidx_v, msg_v, bounce_v, y_v, acc_sh):
    c = lax.axis_index("c")
    s = lax.axis_index("s")
    wid = c * NS + s

    pltpu.sync_copy(y_hbm, y_v)

    _zero_vmem(bounce_v, SLICE)
    pltpu.sync_copy(bounce_v, acc_sh.at[pl.ds(s * SLICE, SLICE)])
    plsc.subcore_barrier()

    row0 = wid * ROWS_PT
    flat0 = row0 * 128

    def macro(m, _):
        pltpu.sync_copy(src_hbm.at[pl.ds(flat0 + m * (R * 128), R * 128)],
                        src_v)
        pltpu.sync_copy(dst_hbm.at[pl.ds(row0 + m * R, R)], dstidx_v)
        for j in range(R):
            for k in range(128 // L):
                idx = src_v[pl.ds(j * 128 + k * L, L)]
                msg_v[pl.ds(k * L, L)] = plsc.load_gather(y_v, [idx])
            pltpu.sync_copy(msg_v, acc_sh.at[dstidx_v.at[j]], add=True)
        return 0

    lax.fori_loop(0, NMACRO, macro, 0)
    plsc.subcore_barrier()

    pltpu.sync_copy(acc_sh.at[pl.ds(s * SLICE, SLICE)], bounce_v)
    pltpu.sync_copy(bounce_v, out_hbm.at[wid])


def _node_prep_body(x0, x1, x2, da, db, w_ref, y, dis, selfterm):
    deg = da[:, :] + db[:, :] + 1.0
    d = lax.rsqrt(deg)
    xw = (x0[:, :] * w_ref[0, 0] + x1[:, :] * w_ref[1, 0]
          + x2[:, :] * w_ref[2, 0])
    dis[:, :] = d
    y[:, :] = xw * d
    selfterm[:, :] = xw / deg


def _combine_body(s0, s1, dis, selfterm, b_ref, out):
    out[:, :] = dis[:, :] * (s0[:, :] + s1[:, :]) + selfterm[:, :] + b_ref[0, 0]


def kernel(x, edge_index, W, b):
    n = x.shape[0]
    e = edge_index.shape[1]

    ei = edge_index.astype(jnp.int32)
    src = jnp.pad(ei[0], (0, EPAD - e), constant_values=PAD_NODE)
    dst = jnp.pad(ei[1], (0, EPAD - e), constant_values=PAD_NODE)
    dst2d = dst.reshape(ROWS, 128)

    xt = jnp.pad(x, ((0, NPAD - n), (0, 0))).T
    x0 = xt[0].reshape(NPAD // 128, 128)
    x1 = xt[1].reshape(NPAD // 128, 128)
    x2 = xt[2].reshape(NPAD // 128, 128)

    degparts = _degree_kernel(dst2d)
    dp = degparts.reshape(NC, NPAD // 128, 128)

    grid2d = (NPAD // 128, 128)
    vspec = pl.BlockSpec(memory_space=pltpu.VMEM)
    sspec = pl.BlockSpec(memory_space=pltpu.SMEM)
    y, dis, selfterm = pl.pallas_call(
        _node_prep_body,
        out_shape=[jax.ShapeDtypeStruct(grid2d, jnp.float32)] * 3,
        in_specs=[vspec] * 5 + [sspec],
    )(x0, x1, x2, dp[0], dp[1], W)

    sparts = _scatter_kernel(src, dst2d, y.reshape(NPAD))
    sp = sparts.reshape(NC, NPAD // 128, 128)

    out2d = pl.pallas_call(
        _combine_body,
        out_shape=jax.ShapeDtypeStruct(grid2d, jnp.float32),
        in_specs=[vspec] * 4 + [sspec],
    )(sp[0], sp[1], dis, selfterm, b.reshape(1, 1))

    return out2d.reshape(NPAD, 1)[:n]


# speedup vs baseline: 438.2762x; 2.0179x over previous
import functools

import jax
import jax.numpy as jnp
from jax import lax
from jax.experimental import pallas as pl
from jax.experimental.pallas import tpu as pltpu
from jax.experimental.pallas import tpu_sc as plsc

NC = 2
NS = 16
NW = NC * NS
L = 16

NPAD = 102400
SLICE = NPAD // NS
ROWS_PT = 1568
ROWS = ROWS_PT * NW
EPAD = ROWS * 128
R = 16
NMACRO = ROWS_PT // R
NPAIR = NMACRO // 2
PAD_NODE = NPAD - 1

_mesh = plsc.VectorSubcoreMesh(core_axis_name="c", subcore_axis_name="s")
_sc_params = pltpu.CompilerParams(needs_layout_passes=False,
                                  use_tc_tiling_on_sc=False)


def _fill_vmem(buf, nwords, vec):
    def body(i, _):
        buf[pl.ds(i * vec.shape[0], vec.shape[0])] = vec
        return 0

    lax.fori_loop(0, nwords // vec.shape[0], body, 0)


@functools.partial(
    pl.kernel,
    out_type=jax.ShapeDtypeStruct((NW * SLICE,), jnp.float32),
    mesh=_mesh,
    scratch_types=[
        pltpu.VMEM((R, 128), jnp.int32),
        pltpu.VMEM((R, 128), jnp.int32),
        pltpu.VMEM((128,), jnp.float32),
        pltpu.VMEM((SLICE,), jnp.float32),
        pltpu.VMEM_SHARED((NPAD,), jnp.float32),
        pltpu.SemaphoreType.DMA,
        pltpu.SemaphoreType.DMA,
        pltpu.SemaphoreType.DMA,
        pltpu.SemaphoreType.DMA,
    ],
    compiler_params=_sc_params,
)
def _degree_kernel(dst_hbm, ones_hbm, zeros_hbm, out_hbm, idx_a, idx_b,
                   ones_v, bounce_v, acc_sh, sem_a, sem_b, sem_sa, sem_sb):
    c = lax.axis_index("c")
    s = lax.axis_index("s")
    wid = c * NS + s

    pltpu.sync_copy(ones_hbm, ones_v)
    pltpu.sync_copy(zeros_hbm, acc_sh.at[pl.ds(s * SLICE, SLICE)])
    plsc.subcore_barrier()

    row0 = wid * ROWS_PT
    pltpu.async_copy(dst_hbm.at[pl.ds(row0, R)], idx_a, sem_a)
    pltpu.async_copy(dst_hbm.at[pl.ds(row0 + R, R)], idx_b, sem_b)

    def pair(t, _):
        ra = row0 + 2 * t * R
        pltpu.make_async_copy(dst_hbm.at[pl.ds(ra, R)], idx_a, sem_a).wait()
        da = [pltpu.async_copy(ones_v, acc_sh.at[idx_a.at[j]], sem_sa,
                               add=True) for j in range(R)]
        pltpu.make_async_copy(dst_hbm.at[pl.ds(ra + R, R)], idx_b,
                              sem_b).wait()
        db = [pltpu.async_copy(ones_v, acc_sh.at[idx_b.at[j]], sem_sb,
                               add=True) for j in range(R)]
        for d in da:
            d.wait()

        @pl.when(t < NPAIR - 1)
        def _():
            pltpu.async_copy(dst_hbm.at[pl.ds(ra + 2 * R, R)], idx_a, sem_a)

        for d in db:
            d.wait()

        @pl.when(t < NPAIR - 1)
        def _():
            pltpu.async_copy(dst_hbm.at[pl.ds(ra + 3 * R, R)], idx_b, sem_b)

        return 0

    lax.fori_loop(0, NPAIR, pair, 0)
    plsc.subcore_barrier()

    pltpu.sync_copy(acc_sh.at[pl.ds(s * SLICE, SLICE)], bounce_v)
    pltpu.sync_copy(bounce_v, out_hbm.at[pl.ds(wid * SLICE, SLICE)])


@functools.partial(
    pl.kernel,
    out_type=jax.ShapeDtypeStruct((NW * SLICE,), jnp.float32),
    mesh=_mesh,
    scratch_types=[
        pltpu.VMEM((R * 128,), jnp.int32),
        pltpu.VMEM((R * 128,), jnp.int32),
        pltpu.VMEM((R, 128), jnp.int32),
        pltpu.VMEM((R, 128), jnp.int32),
        pltpu.VMEM((R, 128), jnp.float32),
        pltpu.VMEM((R, 128), jnp.float32),
        pltpu.VMEM((SLICE,), jnp.float32),
        pltpu.VMEM((NPAD,), jnp.float32),
        pltpu.VMEM_SHARED((NPAD,), jnp.float32),
        pltpu.SemaphoreType.DMA,
        pltpu.SemaphoreType.DMA,
        pltpu.SemaphoreType.DMA,
        pltpu.SemaphoreType.DMA,
    ],
    compiler_params=_sc_params,
)
def _scatter_kernel(src_hbm, dst_hbm, y_hbm, out_hbm,
                    src_a, src_b, dst_a, dst_b, msg_a, msg_b,
                    bounce_v, y_v, acc_sh, sem_a, sem_b, sem_sa, sem_sb):
    c = lax.axis_index("c")
    s = lax.axis_index("s")
    wid = c * NS + s

    pltpu.sync_copy(y_hbm, y_v)
    _fill_vmem(bounce_v, SLICE, jnp.zeros((16,), jnp.float32))
    pltpu.sync_copy(bounce_v, acc_sh.at[pl.ds(s * SLICE, SLICE)])
    plsc.subcore_barrier()

    row0 = wid * ROWS_PT
    flat0 = row0 * 128
    CH = R * 128

    def stage(m, src_v, dst_v, sem):
        pltpu.async_copy(src_hbm.at[pl.ds(flat0 + m * CH, CH)], src_v, sem)
        pltpu.async_copy(dst_hbm.at[pl.ds(row0 + m * R, R)], dst_v, sem)

    def wait_stage(m, src_v, dst_v, sem):
        pltpu.make_async_copy(src_hbm.at[pl.ds(flat0 + m * CH, CH)], src_v,
                              sem).wait()
        pltpu.make_async_copy(dst_hbm.at[pl.ds(row0 + m * R, R)], dst_v,
                              sem).wait()

    def gather_streams(src_v, dst_v, msg_v, sem_s):
        descs = []
        for j in range(R):
            for k in range(128 // L):
                idx = src_v[pl.ds(j * 128 + k * L, L)]
                msg_v[j, pl.ds(k * L, L)] = plsc.load_gather(y_v, [idx])
            descs.append(pltpu.async_copy(msg_v.at[j],
                                          acc_sh.at[dst_v.at[j]],
                                          sem_s, add=True))
        return descs

    stage(0, src_a, dst_a, sem_a)
    stage(1, src_b, dst_b, sem_b)

    def pair(t, _):
        m = 2 * t
        wait_stage(m, src_a, dst_a, sem_a)
        da = gather_streams(src_a, dst_a, msg_a, sem_sa)
        wait_stage(m + 1, src_b, dst_b, sem_b)
        db = gather_streams(src_b, dst_b, msg_b, sem_sb)
        for d in da:
            d.wait()

        @pl.when(t < NPAIR - 1)
        def _():
            stage(m + 2, src_a, dst_a, sem_a)

        for d in db:
            d.wait()

        @pl.when(t < NPAIR - 1)
        def _():
            stage(m + 3, src_b, dst_b, sem_b)

        return 0

    lax.fori_loop(0, NPAIR, pair, 0)
    plsc.subcore_barrier()

    pltpu.sync_copy(acc_sh.at[pl.ds(s * SLICE, SLICE)], bounce_v)
    pltpu.sync_copy(bounce_v, out_hbm.at[pl.ds(wid * SLICE, SLICE)])


def _node_prep_body(x0, x1, x2, da, db, w_ref, y, dis, selfterm):
    deg = da[:, :] + db[:, :] + 1.0
    d = lax.rsqrt(deg)
    xw = (x0[:, :] * w_ref[0, 0] + x1[:, :] * w_ref[1, 0]
          + x2[:, :] * w_ref[2, 0])
    dis[:, :] = d
    y[:, :] = xw * d
    selfterm[:, :] = xw / deg


def _combine_body(s0, s1, dis, selfterm, b_ref, out):
    out[:, :] = dis[:, :] * (s0[:, :] + s1[:, :]) + selfterm[:, :] + b_ref[0, 0]


def kernel(x, edge_index, W, b):
    n = x.shape[0]
    e = edge_index.shape[1]

    ei = edge_index.astype(jnp.int32)
    src = jnp.pad(ei[0], (0, EPAD - e), constant_values=PAD_NODE)
    dst = jnp.pad(ei[1], (0, EPAD - e), constant_values=PAD_NODE)
    dst2d = dst.reshape(ROWS, 128)

    xt = jnp.pad(x, ((0, NPAD - n), (0, 0))).T
    x0 = xt[0].reshape(NPAD // 128, 128)
    x1 = xt[1].reshape(NPAD // 128, 128)
    x2 = xt[2].reshape(NPAD // 128, 128)

    ones_c = jnp.ones((128,), jnp.float32)
    zeros_c = jnp.zeros((SLICE,), jnp.float32)
    degparts = _degree_kernel(dst2d, ones_c, zeros_c)
    dp = degparts.reshape(NC, NPAD // 128, 128)

    grid2d = (NPAD // 128, 128)
    vspec = pl.BlockSpec(memory_space=pltpu.VMEM)
    sspec = pl.BlockSpec(memory_space=pltpu.SMEM)
    y, dis, selfterm = pl.pallas_call(
        _node_prep_body,
        out_shape=[jax.ShapeDtypeStruct(grid2d, jnp.float32)] * 3,
        in_specs=[vspec] * 5 + [sspec],
    )(x0, x1, x2, dp[0], dp[1], W)

    sparts = _scatter_kernel(src, dst2d, y.reshape(NPAD))
    sp = sparts.reshape(NC, NPAD // 128, 128)

    out2d = pl.pallas_call(
        _combine_body,
        out_shape=jax.ShapeDtypeStruct(grid2d, jnp.float32),
        in_specs=[vspec] * 4 + [sspec],
    )(sp[0], sp[1], dis, selfterm, b.reshape(1, 1))

    return out2d.reshape(NPAD, 1)[:n]


# speedup vs baseline: 530.0408x; 1.2094x over previous
import functools

import jax
import jax.numpy as jnp
from jax import lax
from jax.experimental import pallas as pl
from jax.experimental.pallas import tpu as pltpu
from jax.experimental.pallas import tpu_sc as plsc

NC = 2
NS = 16
NW = NC * NS
L = 16

NPAD = 102400
SLICE = NPAD // NS
E_ROWS = 50000
R = 16
NCHUNKS = E_ROWS // R
CPW = NCHUNKS // NW
EXTRA = NCHUNKS - CPW * NW

_mesh = plsc.VectorSubcoreMesh(core_axis_name="c", subcore_axis_name="s")
_sc_params = pltpu.CompilerParams(needs_layout_passes=False,
                                  use_tc_tiling_on_sc=False)


def _fill_vmem(buf, nwords, vec):
    def body(i, _):
        buf[pl.ds(i * vec.shape[0], vec.shape[0])] = vec
        return 0

    lax.fori_loop(0, nwords // vec.shape[0], body, 0)


def _worker_chunks(wid):
    c0 = wid * CPW + jnp.minimum(wid, EXTRA)
    nch = CPW + (wid < EXTRA).astype(jnp.int32)
    return c0, nch


@functools.partial(
    pl.kernel,
    out_type=jax.ShapeDtypeStruct((NW * SLICE,), jnp.float32),
    mesh=_mesh,
    scratch_types=[
        pltpu.VMEM((R, 128), jnp.int32),
        pltpu.VMEM((R, 128), jnp.int32),
        pltpu.VMEM((128,), jnp.float32),
        pltpu.VMEM((SLICE,), jnp.float32),
        pltpu.VMEM_SHARED((NPAD,), jnp.float32),
        pltpu.SemaphoreType.DMA,
        pltpu.SemaphoreType.DMA,
        pltpu.SemaphoreType.DMA,
        pltpu.SemaphoreType.DMA,
    ],
    compiler_params=_sc_params,
)
def _degree_kernel(edge_hbm, out_hbm, idx_a, idx_b, ones_v, bounce_v, acc_sh,
                   sem_a, sem_b, sem_sa, sem_sb):
    c = lax.axis_index("c")
    s = lax.axis_index("s")
    wid = c * NS + s

    _fill_vmem(ones_v, 128, jnp.ones((L,), jnp.float32))
    _fill_vmem(bounce_v, SLICE, jnp.zeros((L,), jnp.float32))
    pltpu.sync_copy(bounce_v, acc_sh.at[pl.ds(s * SLICE, SLICE)])
    plsc.subcore_barrier()

    c0, nch = _worker_chunks(wid)
    npair = nch // 2
    odd = nch - 2 * npair

    def stage(ch, buf, sem):
        pltpu.async_copy(edge_hbm.at[1, pl.ds(R * ch, R)], buf, sem)

    def wait_stage(ch, buf, sem):
        pltpu.make_async_copy(edge_hbm.at[1, pl.ds(R * ch, R)], buf,
                              sem).wait()

    def streams(buf, sem_s):
        return [pltpu.async_copy(ones_v, acc_sh.at[buf.at[j]], sem_s,
                                 add=True) for j in range(R)]

    stage(c0, idx_a, sem_a)
    stage(c0 + 1, idx_b, sem_b)

    def pair(t, _):
        ca = c0 + 2 * t
        wait_stage(ca, idx_a, sem_a)
        da = streams(idx_a, sem_sa)
        wait_stage(ca + 1, idx_b, sem_b)
        db = streams(idx_b, sem_sb)
        for d in da:
            d.wait()

        @pl.when(2 * t + 2 < nch)
        def _():
            stage(ca + 2, idx_a, sem_a)

        for d in db:
            d.wait()

        @pl.when(2 * t + 3 < nch)
        def _():
            stage(ca + 3, idx_b, sem_b)

        return 0

    lax.fori_loop(0, npair, pair, 0)

    @pl.when(odd == 1)
    def _():
        wait_stage(c0 + 2 * npair, idx_a, sem_a)
        for d in streams(idx_a, sem_sa):
            d.wait()

    plsc.subcore_barrier()

    pltpu.sync_copy(acc_sh.at[pl.ds(s * SLICE, SLICE)], bounce_v)
    pltpu.sync_copy(bounce_v, out_hbm.at[pl.ds(wid * SLICE, SLICE)])


@functools.partial(
    pl.kernel,
    out_type=jax.ShapeDtypeStruct((NW * SLICE,), jnp.float32),
    mesh=_mesh,
    scratch_types=[
        pltpu.VMEM((R, 128), jnp.int32),
        pltpu.VMEM((R, 128), jnp.int32),
        pltpu.VMEM((R, 128), jnp.int32),
        pltpu.VMEM((R, 128), jnp.int32),
        pltpu.VMEM((R, 128), jnp.float32),
        pltpu.VMEM((R, 128), jnp.float32),
        pltpu.VMEM((SLICE,), jnp.float32),
        pltpu.VMEM((NPAD,), jnp.float32),
        pltpu.VMEM_SHARED((NPAD,), jnp.float32),
        pltpu.SemaphoreType.DMA,
        pltpu.SemaphoreType.DMA,
        pltpu.SemaphoreType.DMA,
        pltpu.SemaphoreType.DMA,
    ],
    compiler_params=_sc_params,
)
def _scatter_kernel(edge_hbm, y_hbm, out_hbm,
                    src_a, src_b, dst_a, dst_b, msg_a, msg_b,
                    bounce_v, y_v, acc_sh, sem_a, sem_b, sem_sa, sem_sb):
    c = lax.axis_index("c")
    s = lax.axis_index("s")
    wid = c * NS + s

    pltpu.sync_copy(y_hbm, y_v)
    _fill_vmem(bounce_v, SLICE, jnp.zeros((L,), jnp.float32))
    pltpu.sync_copy(bounce_v, acc_sh.at[pl.ds(s * SLICE, SLICE)])
    plsc.subcore_barrier()

    c0, nch = _worker_chunks(wid)
    npair = nch // 2
    odd = nch - 2 * npair

    def stage(ch, src_v, dst_v, sem):
        pltpu.async_copy(edge_hbm.at[0, pl.ds(R * ch, R)], src_v, sem)
        pltpu.async_copy(edge_hbm.at[1, pl.ds(R * ch, R)], dst_v, sem)

    def wait_stage(ch, src_v, dst_v, sem):
        pltpu.make_async_copy(edge_hbm.at[0, pl.ds(R * ch, R)], src_v,
                              sem).wait()
        pltpu.make_async_copy(edge_hbm.at[1, pl.ds(R * ch, R)], dst_v,
                              sem).wait()

    def gather_streams(src_v, dst_v, msg_v, sem_s):
        descs = []
        for j in range(R):
            for k in range(128 // L):
                idx = src_v[j, pl.ds(k * L, L)]
                msg_v[j, pl.ds(k * L, L)] = plsc.load_gather(y_v, [idx])
            descs.append(pltpu.async_copy(msg_v.at[j],
                                          acc_sh.at[dst_v.at[j]],
                                          sem_s, add=True))
        return descs

    stage(c0, src_a, dst_a, sem_a)
    stage(c0 + 1, src_b, dst_b, sem_b)

    def pair(t, _):
        ca = c0 + 2 * t
        wait_stage(ca, src_a, dst_a, sem_a)
        da = gather_streams(src_a, dst_a, msg_a, sem_sa)
        wait_stage(ca + 1, src_b, dst_b, sem_b)
        db = gather_streams(src_b, dst_b, msg_b, sem_sb)
        for d in da:
            d.wait()

        @pl.when(2 * t + 2 < nch)
        def _():
            stage(ca + 2, src_a, dst_a, sem_a)

        for d in db:
            d.wait()

        @pl.when(2 * t + 3 < nch)
        def _():
            stage(ca + 3, src_b, dst_b, sem_b)

        return 0

    lax.fori_loop(0, npair, pair, 0)

    @pl.when(odd == 1)
    def _():
        wait_stage(c0 + 2 * npair, src_a, dst_a, sem_a)
        for d in gather_streams(src_a, dst_a, msg_a, sem_sa):
            d.wait()

    plsc.subcore_barrier()

    pltpu.sync_copy(acc_sh.at[pl.ds(s * SLICE, SLICE)], bounce_v)
    pltpu.sync_copy(bounce_v, out_hbm.at[pl.ds(wid * SLICE, SLICE)])


def _node_prep_body(x0, x1, x2, da, db, w_ref, y, dis, selfterm):
    deg = da[:, :] + db[:, :] + 1.0
    d = lax.rsqrt(deg)
    xw = (x0[:, :] * w_ref[0, 0] + x1[:, :] * w_ref[1, 0]
          + x2[:, :] * w_ref[2, 0])
    dis[:, :] = d
    y[:, :] = xw * d
    selfterm[:, :] = xw / deg


def _combine_body(s0, s1, dis, selfterm, b_ref, out):
    out[:, :] = dis[:, :] * (s0[:, :] + s1[:, :]) + selfterm[:, :] + b_ref[0, 0]


def kernel(x, edge_index, W, b):
    n = x.shape[0]

    edge3d = edge_index.astype(jnp.int32).reshape(2, E_ROWS, 128)

    xt = jnp.pad(x, ((0, NPAD - n), (0, 0))).T
    x0 = xt[0].reshape(NPAD // 128, 128)
    x1 = xt[1].reshape(NPAD // 128, 128)
    x2 = xt[2].reshape(NPAD // 128, 128)

    degparts = _degree_kernel(edge3d)
    dp = degparts.reshape(NC, NPAD // 128, 128)

    grid2d = (NPAD // 128, 128)
    vspec = pl.BlockSpec(memory_space=pltpu.VMEM)
    sspec = pl.BlockSpec(memory_space=pltpu.SMEM)
    y, dis, selfterm = pl.pallas_call(
        _node_prep_body,
        out_shape=[jax.ShapeDtypeStruct(grid2d, jnp.float32)] * 3,
        in_specs=[vspec] * 5 + [sspec],
    )(x0, x1, x2, dp[0], dp[1], W)

    sparts = _scatter_kernel(edge3d, y.reshape(NPAD))
    sp = sparts.reshape(NC, NPAD // 128, 128)

    out2d = pl.pallas_call(
        _combine_body,
        out_shape=jax.ShapeDtypeStruct(grid2d, jnp.float32),
        in_specs=[vspec] * 4 + [sspec],
    )(sp[0], sp[1], dis, selfterm, b.reshape(1, 1))

    return out2d.reshape(NPAD, 1)[:n]
